# baseline (speedup 1.0000x reference)
import functools

import jax
import jax.numpy as jnp
from jax import lax
from jax.experimental import pallas as pl
from jax.experimental.pallas import tpu as pltpu
from jax.experimental.pallas import tpu_sc as plsc

D = 128
HEADS = 4
N_L = 10000
N_G = 50000
N_D = 10000
E = 100000

NSUB = 16
BATCH = 128
EPAD = 102400
EROWS = EPAD // BATCH
NPS = 10240
NPG = 50176


def _mesh():
    return plsc.VectorSubcoreMesh(core_axis_name="c", subcore_axis_name="s")


def _fill16(ref, n, val):
    @pl.loop(0, n)
    def _(i):
        ref[i, :] = val


def _seg_full_x4(rels):
    rows_per = NPS // NSUB
    nbatch = EROWS // 32

    @functools.partial(
        pl.kernel,
        mesh=_mesh(),
        out_type=tuple(jax.ShapeDtypeStruct((2, NPS, D), jnp.float32)
                       for _ in range(4)),
        scratch_types=[
            pltpu.VMEM_SHARED((NPS, D), jnp.float32),
            pltpu.VMEM((nbatch, 1, BATCH), jnp.int32),
            pltpu.VMEM((nbatch, 1, BATCH), jnp.int32),
            pltpu.VMEM((BATCH, D), jnp.float32),
            pltpu.VMEM((16, D), jnp.float32),
            pltpu.SemaphoreType.DMA,
        ],
    )
    def k(t0, s0, d0, t1, s1, d1, t2, s2, d2, t3, s3, d3,
          o0, o1, o2, o3, acc, src_v, dst_v, rows_v, zb, sem):
        c = lax.axis_index("c")
        s = lax.axis_index("s")
        w = c * NSUB + s
        zero16 = jnp.zeros((16,), jnp.float32)

        @pl.loop(0, 16)
        def _(i):
            for j in range(D // 16):
                zb[i, pl.ds(j * 16, 16)] = zero16

        r0 = s * rows_per
        for tbl, srcr, dstr, outr in ((t0, s0, d0, o0), (t1, s1, d1, o1),
                                      (t2, s2, d2, o2), (t3, s3, d3, o3)):
            @pl.loop(0, rows_per // 16)
            def _(t):
                pltpu.sync_copy(zb, acc.at[pl.ds(r0 + t * 16, 16)])

            pltpu.sync_copy(srcr.at[pl.ds(w * nbatch, nbatch)], src_v)
            pltpu.sync_copy(dstr.at[pl.ds(w * nbatch, nbatch)], dst_v)
            plsc.subcore_barrier()

            @pl.loop(0, nbatch)
            def _(j):
                pltpu.async_copy(tbl.at[src_v.at[j, 0]], rows_v, sem).wait()
                pltpu.sync_copy(rows_v, acc.at[dst_v.at[j, 0]], add=True)

            plsc.subcore_barrier()
            pltpu.sync_copy(acc.at[pl.ds(r0, rows_per)],
                            outr.at[c, pl.ds(r0, rows_per)])

    return k(*[x for rel in rels for x in rel])


def _seg_chunk(table4, src3d, dst3d):
    rows_per = NPG // NSUB
    nbatch = EROWS // NSUB

    @functools.partial(
        pl.kernel,
        mesh=_mesh(),
        compiler_params=pltpu.CompilerParams(use_tc_tiling_on_sc=False),
        out_type=jax.ShapeDtypeStruct((4, NPG, 32), jnp.float32),
        scratch_types=[
            pltpu.VMEM_SHARED((NPG, 32), jnp.float32),
            pltpu.VMEM((nbatch, 1, BATCH), jnp.int32),
            pltpu.VMEM((nbatch, 1, BATCH), jnp.int32),
            pltpu.VMEM((nbatch, 1, BATCH), jnp.int32),
            pltpu.VMEM((BATCH, 32), jnp.float32),
            pltpu.VMEM((16, 32), jnp.float32),
            pltpu.SemaphoreType.DMA,
        ],
    )
    def k(tbl, src, dst, out_s, acc, src_v, sidx_v, dst_v, rows_v, zb32, sem):
        c = lax.axis_index("c")
        s = lax.axis_index("s")
        zero16 = jnp.zeros((16,), jnp.float32)

        @pl.loop(0, 16)
        def _(i):
            zb32[i, pl.ds(0, 16)] = zero16
            zb32[i, pl.ds(16, 16)] = zero16

        r0 = s * rows_per
        pltpu.sync_copy(src.at[pl.ds(s * nbatch, nbatch)], src_v)
        pltpu.sync_copy(dst.at[pl.ds(s * nbatch, nbatch)], dst_v)

        for k2 in range(2):
            chunk = c * 2 + k2

            @pl.loop(0, rows_per // 16)
            def _(t):
                pltpu.sync_copy(zb32, acc.at[pl.ds(r0 + t * 16, 16)])

            @pl.loop(0, nbatch)
            def _(i):
                for j in range(BATCH // 16):
                    sidx_v[i, 0, pl.ds(j * 16, 16)] = (
                        src_v[i, 0, pl.ds(j * 16, 16)] * 4 + chunk)

            plsc.subcore_barrier()

            @pl.loop(0, nbatch)
            def _(j):
                pltpu.async_copy(tbl.at[sidx_v.at[j, 0]], rows_v, sem).wait()
                pltpu.sync_copy(rows_v, acc.at[dst_v.at[j, 0]], add=True)

            plsc.subcore_barrier()
            pltpu.sync_copy(acc.at[pl.ds(r0, rows_per)],
                            out_s.at[chunk, pl.ds(r0, rows_per)])

    return k(table4, src3d, dst3d)


def _deg_x6(dsts_small, dsts_gene):
    rps = NPS // NSUB
    rpg = NPG // NSUB
    nbatch = EROWS // NSUB

    @functools.partial(
        pl.kernel,
        mesh=_mesh(),
        compiler_params=pltpu.CompilerParams(use_tc_tiling_on_sc=False),
        out_type=(jax.ShapeDtypeStruct((4, NPS, 16), jnp.float32),
                  jax.ShapeDtypeStruct((2, NPG, 16), jnp.float32)),
        scratch_types=[
            pltpu.VMEM_SHARED((NPG, 16), jnp.float32),
            pltpu.VMEM((nbatch, 1, BATCH), jnp.int32),
            pltpu.VMEM((BATCH, 16), jnp.float32),
            pltpu.VMEM((16, 16), jnp.float32),
        ],
    )
    def k(dsml, dgen, out_s, out_g, dacc, dst_v, ones_v, zb16):
        c = lax.axis_index("c")
        s = lax.axis_index("s")
        _fill16(zb16, 16, jnp.zeros((16,), jnp.float32))
        _fill16(ones_v, BATCH, jnp.ones((16,), jnp.float32))

        for (inref, outref, nrel_pc, rows_per) in (
                (dsml, out_s, 2, rps), (dgen, out_g, 1, rpg)):
            r0 = s * rows_per
            for rr in range(nrel_pc):
                rel = c * nrel_pc + rr

                @pl.loop(0, rows_per // 16)
                def _(t):
                    pltpu.sync_copy(zb16, dacc.at[pl.ds(r0 + t * 16, 16)])

                pltpu.sync_copy(inref.at[rel, pl.ds(s * nbatch, nbatch)],
                                dst_v)
                plsc.subcore_barrier()

                @pl.loop(0, nbatch)
                def _(j):
                    pltpu.sync_copy(ones_v, dacc.at[dst_v.at[j, 0]],
                                    add=True)

                plsc.subcore_barrier()
                pltpu.sync_copy(dacc.at[pl.ds(r0, rows_per)],
                                outref.at[rel, pl.ds(r0, rows_per)])

    return k(dsts_small, dsts_gene)


_ATT_SCALE = 1.0 / (float(D) ** 0.5 * 0.5)


def _update_body(partials, h_ref, s0_ref, d0_ref, s1_ref, d1_ref,
                 w0_ref, w1_ref, ws_ref, a0_ref, a1_ref, g_ref, b_ref, o_ref):
    if partials:
        S0 = s0_ref[0] + s0_ref[1]
        S1 = s1_ref[0] + s1_ref[1]
    else:
        S0 = jnp.concatenate([s0_ref[k] for k in range(4)], axis=-1)
        S1 = jnp.concatenate([s1_ref[k] for k in range(4)], axis=-1)
    dg0 = d0_ref[:, 0:1]
    dg1 = d1_ref[:, 0:1]
    dg0 = jnp.where(dg0 == 0.0, 1.0, dg0)
    dg1 = jnp.where(dg1 == 0.0, 1.0, dg1)
    m0 = jnp.dot(S0, w0_ref[...], preferred_element_type=jnp.float32) / dg0
    m1 = jnp.dot(S1, w1_ref[...], preferred_element_type=jnp.float32) / dg1
    dn = (((1,), (1,)), ((), ()))
    sc0 = lax.dot_general(m0, a0_ref[...], dn,
                          preferred_element_type=jnp.float32)[:, :HEADS]
    sc1 = lax.dot_general(m1, a1_ref[...], dn,
                          preferred_element_type=jnp.float32)[:, :HEADS]
    sc0 = sc0 * _ATT_SCALE
    sc1 = sc1 * _ATT_SCALE
    mx = jnp.maximum(sc0, sc1)
    e0 = jnp.exp(sc0 - mx)
    e1 = jnp.exp(sc1 - mx)
    wt0 = jnp.mean(e0 / (e0 + e1), axis=1, keepdims=True)
    wt1 = 1.0 - wt0
    agg = wt0 * m0 + wt1 * m1
    hb = h_ref[...]
    h_up = jnp.dot(hb, ws_ref[...], preferred_element_type=jnp.float32) + agg + hb
    mu = jnp.mean(h_up, axis=1, keepdims=True)
    cen = h_up - mu
    var = jnp.mean(cen * cen, axis=1, keepdims=True)
    hn = cen / jnp.sqrt(var + 1e-5) * g_ref[...] + b_ref[...]
    o_ref[...] = jnp.where(hn >= 0.0, hn, 0.01 * hn)


def _update_call(partials, n, h, S0, D0, S1, D1, W0, W1, Ws, a0p, a1p, g2, b2):
    bn = 400
    if partials:
        s_spec = pl.BlockSpec((2, bn, D), lambda i: (0, i, 0))
    else:
        s_spec = pl.BlockSpec((4, bn, 32), lambda i: (0, i, 0))
    d_spec = pl.BlockSpec((bn, 16), lambda i: (i, 0))
    wspec = pl.BlockSpec((D, D), lambda i: (0, 0))
    aspec = pl.BlockSpec((8, D), lambda i: (0, 0))
    vspec = pl.BlockSpec((1, D), lambda i: (0, 0))
    return pl.pallas_call(
        functools.partial(_update_body, partials),
        grid=(n // bn,),
        in_specs=[pl.BlockSpec((bn, D), lambda i: (i, 0)),
                  s_spec, d_spec, s_spec, d_spec,
                  wspec, wspec, wspec, aspec, aspec, vspec, vspec],
        out_specs=pl.BlockSpec((bn, D), lambda i: (i, 0)),
        out_shape=jax.ShapeDtypeStruct((n, D), jnp.float32),
    )(h, S0, D0, S1, D1, W0, W1, Ws, a0p, a1p, g2, b2)


def _pad_edges(src, dst, n_dst, npad):
    pad_dst = n_dst + jnp.arange(EPAD - E, dtype=jnp.int32) % (npad - n_dst)
    src_p = jnp.concatenate(
        [src.astype(jnp.int32), jnp.zeros((EPAD - E,), jnp.int32)])
    dst_p = jnp.concatenate([dst.astype(jnp.int32), pad_dst])
    return src_p.reshape(EROWS, 1, BATCH), dst_p.reshape(EROWS, 1, BATCH)


def _pad_att(a):
    return jnp.pad(a, ((0, 8 - HEADS), (0, 0)))


def kernel(h_lncRNA, h_gene, h_disease, src_lg, dst_lg, src_dg, dst_dg,
           src_gd, dst_gd, src_ld, dst_ld, src_gl, dst_gl, src_dl, dst_dl,
           W_lg, W_dg, W_gd, W_ld, W_gl, W_dl, Ws_l, Ws_g, Ws_d,
           a_lg, a_dg, a_gd, a_ld, a_gl, a_dl,
           ln_g_l, ln_b_l, ln_g_g, ln_b_g, ln_g_d, ln_b_d):
    s_gd, d_gd = _pad_edges(src_gd, dst_gd, N_D, NPS)
    s_ld, d_ld = _pad_edges(src_ld, dst_ld, N_D, NPS)
    s_gl, d_gl = _pad_edges(src_gl, dst_gl, N_L, NPS)
    s_dl, d_dl = _pad_edges(src_dl, dst_dl, N_L, NPS)
    s_lg, d_lg = _pad_edges(src_lg, dst_lg, N_G, NPG)
    s_dg, d_dg = _pad_edges(src_dg, dst_dg, N_G, NPG)

    S_gd, S_ld, S_gl, S_dl = _seg_full_x4([
        (h_gene, s_gd, d_gd), (h_lncRNA, s_ld, d_ld),
        (h_gene, s_gl, d_gl), (h_disease, s_dl, d_dl)])
    S_lg = _seg_chunk(h_lncRNA.reshape(4 * N_L, 32), s_lg, d_lg)
    S_dg = _seg_chunk(h_disease.reshape(4 * N_D, 32), s_dg, d_dg)
    Dg_s, Dg_g = _deg_x6(jnp.stack([d_gd, d_ld, d_gl, d_dl]),
                         jnp.stack([d_lg, d_dg]))
    Dg_gd, Dg_ld, Dg_gl, Dg_dl = Dg_s[0], Dg_s[1], Dg_s[2], Dg_s[3]
    Dg_lg, Dg_dg = Dg_g[0], Dg_g[1]

    g_l, b_l = ln_g_l.reshape(1, D), ln_b_l.reshape(1, D)
    g_g, b_g = ln_g_g.reshape(1, D), ln_b_g.reshape(1, D)
    g_d, b_d = ln_g_d.reshape(1, D), ln_b_d.reshape(1, D)
    out_l = _update_call(True, N_L, h_lncRNA, S_gl, Dg_gl, S_dl, Dg_dl,
                         W_gl, W_dl, Ws_l, _pad_att(a_gl), _pad_att(a_dl),
                         g_l, b_l)
    out_g = _update_call(False, N_G, h_gene, S_lg, Dg_lg, S_dg, Dg_dg,
                         W_lg, W_dg, Ws_g, _pad_att(a_lg), _pad_att(a_dg),
                         g_g, b_g)
    out_d = _update_call(True, N_D, h_disease, S_gd, Dg_gd, S_ld, Dg_ld,
                         W_gd, W_ld, Ws_d, _pad_att(a_gd), _pad_att(a_ld),
                         g_d, b_d)
    return (out_l, out_g, out_d)


# speedup vs baseline: 1.0923x; 1.0923x over previous
import functools

import jax
import jax.numpy as jnp
from jax import lax
from jax.experimental import pallas as pl
from jax.experimental.pallas import tpu as pltpu
from jax.experimental.pallas import tpu_sc as plsc

D = 128
HEADS = 4
N_L = 10000
N_G = 50000
N_D = 10000
E = 100000

NSUB = 16
BATCH = 128
EPAD = 102400
EROWS = EPAD // BATCH
NPS = 10240
NPG = 50176


def _mesh():
    return plsc.VectorSubcoreMesh(core_axis_name="c", subcore_axis_name="s")


def _fill16(ref, n, val):
    @pl.loop(0, n)
    def _(i):
        ref[i, :] = val


def _seg_full(table, src3d, dst3d):
    rows_per = NPS // NSUB
    nbatch = EROWS // 32

    @functools.partial(
        pl.kernel,
        mesh=_mesh(),
        out_type=jax.ShapeDtypeStruct((2, NPS, D), jnp.float32),
        scratch_types=[
            pltpu.VMEM_SHARED((NPS, D), jnp.float32),
            pltpu.VMEM((nbatch, 1, BATCH), jnp.int32),
            pltpu.VMEM((nbatch, 1, BATCH), jnp.int32),
            pltpu.VMEM((BATCH, D), jnp.float32),
            pltpu.VMEM((16, D), jnp.float32),
            pltpu.SemaphoreType.DMA,
        ],
    )
    def k(tbl, src, dst, out_sum, acc, src_v, dst_v, rows_v, zb, sem):
        c = lax.axis_index("c")
        s = lax.axis_index("s")
        w = c * NSUB + s
        zero16 = jnp.zeros((16,), jnp.float32)

        @pl.loop(0, 16)
        def _(i):
            for j in range(D // 16):
                zb[i, pl.ds(j * 16, 16)] = zero16

        r0 = s * rows_per

        @pl.loop(0, rows_per // 16)
        def _(t):
            pltpu.sync_copy(zb, acc.at[pl.ds(r0 + t * 16, 16)])

        pltpu.sync_copy(src.at[pl.ds(w * nbatch, nbatch)], src_v)
        pltpu.sync_copy(dst.at[pl.ds(w * nbatch, nbatch)], dst_v)
        plsc.subcore_barrier()

        @pl.loop(0, nbatch)
        def _(j):
            pltpu.async_copy(tbl.at[src_v.at[j, 0]], rows_v, sem).wait()
            pltpu.sync_copy(rows_v, acc.at[dst_v.at[j, 0]], add=True)

        plsc.subcore_barrier()
        pltpu.sync_copy(acc.at[pl.ds(r0, rows_per)],
                        out_sum.at[c, pl.ds(r0, rows_per)])

    return k(table, src3d, dst3d)


def _seg_chunk(table4, src3d, dst3d):
    rows_per = NPG // NSUB
    nbatch = EROWS // NSUB

    @functools.partial(
        pl.kernel,
        mesh=_mesh(),
        compiler_params=pltpu.CompilerParams(use_tc_tiling_on_sc=False),
        out_type=jax.ShapeDtypeStruct((4, NPG, 32), jnp.float32),
        scratch_types=[
            pltpu.VMEM_SHARED((NPG, 32), jnp.float32),
            pltpu.VMEM((nbatch, 1, BATCH), jnp.int32),
            pltpu.VMEM((nbatch, 1, BATCH), jnp.int32),
            pltpu.VMEM((nbatch, 1, BATCH), jnp.int32),
            pltpu.VMEM((BATCH, 32), jnp.float32),
            pltpu.VMEM((16, 32), jnp.float32),
            pltpu.SemaphoreType.DMA,
        ],
    )
    def k(tbl, src, dst, out_s, acc, src_v, sidx_v, dst_v, rows_v, zb32, sem):
        c = lax.axis_index("c")
        s = lax.axis_index("s")
        zero16 = jnp.zeros((16,), jnp.float32)

        @pl.loop(0, 16)
        def _(i):
            zb32[i, pl.ds(0, 16)] = zero16
            zb32[i, pl.ds(16, 16)] = zero16

        r0 = s * rows_per
        pltpu.sync_copy(src.at[pl.ds(s * nbatch, nbatch)], src_v)
        pltpu.sync_copy(dst.at[pl.ds(s * nbatch, nbatch)], dst_v)

        for k2 in range(2):
            chunk = c * 2 + k2

            @pl.loop(0, rows_per // 16)
            def _(t):
                pltpu.sync_copy(zb32, acc.at[pl.ds(r0 + t * 16, 16)])

            @pl.loop(0, nbatch)
            def _(i):
                for j in range(BATCH // 16):
                    sidx_v[i, 0, pl.ds(j * 16, 16)] = (
                        src_v[i, 0, pl.ds(j * 16, 16)] * 4 + chunk)

            plsc.subcore_barrier()

            @pl.loop(0, nbatch)
            def _(j):
                pltpu.async_copy(tbl.at[sidx_v.at[j, 0]], rows_v, sem).wait()
                pltpu.sync_copy(rows_v, acc.at[dst_v.at[j, 0]], add=True)

            plsc.subcore_barrier()
            pltpu.sync_copy(acc.at[pl.ds(r0, rows_per)],
                            out_s.at[chunk, pl.ds(r0, rows_per)])

    return k(table4, src3d, dst3d)


def _deg_x6(dsts_small, dsts_gene):
    rps = NPS // NSUB
    rpg = NPG // NSUB
    nbatch = EROWS // NSUB

    @functools.partial(
        pl.kernel,
        mesh=_mesh(),
        compiler_params=pltpu.CompilerParams(use_tc_tiling_on_sc=False),
        out_type=(jax.ShapeDtypeStruct((4, NPS, 16), jnp.float32),
                  jax.ShapeDtypeStruct((2, NPG, 16), jnp.float32)),
        scratch_types=[
            pltpu.VMEM_SHARED((NPG, 16), jnp.float32),
            pltpu.VMEM((nbatch, 1, BATCH), jnp.int32),
            pltpu.VMEM((BATCH, 16), jnp.float32),
            pltpu.VMEM((16, 16), jnp.float32),
        ],
    )
    def k(dsml, dgen, out_s, out_g, dacc, dst_v, ones_v, zb16):
        c = lax.axis_index("c")
        s = lax.axis_index("s")
        _fill16(zb16, 16, jnp.zeros((16,), jnp.float32))
        _fill16(ones_v, BATCH, jnp.ones((16,), jnp.float32))

        for (inref, outref, nrel_pc, rows_per) in (
                (dsml, out_s, 2, rps), (dgen, out_g, 1, rpg)):
            r0 = s * rows_per
            for rr in range(nrel_pc):
                rel = c * nrel_pc + rr

                @pl.loop(0, rows_per // 16)
                def _(t):
                    pltpu.sync_copy(zb16, dacc.at[pl.ds(r0 + t * 16, 16)])

                pltpu.sync_copy(inref.at[rel, pl.ds(s * nbatch, nbatch)],
                                dst_v)
                plsc.subcore_barrier()

                @pl.loop(0, nbatch)
                def _(j):
                    pltpu.sync_copy(ones_v, dacc.at[dst_v.at[j, 0]],
                                    add=True)

                plsc.subcore_barrier()
                pltpu.sync_copy(dacc.at[pl.ds(r0, rows_per)],
                                outref.at[rel, pl.ds(r0, rows_per)])

    return k(dsts_small, dsts_gene)


_ATT_SCALE = 1.0 / (float(D) ** 0.5 * 0.5)


def _update_body(partials, h_ref, s0_ref, d0_ref, s1_ref, d1_ref,
                 w0_ref, w1_ref, ws_ref, a0_ref, a1_ref, g_ref, b_ref, o_ref):
    if partials:
        S0 = s0_ref[0] + s0_ref[1]
        S1 = s1_ref[0] + s1_ref[1]
    else:
        S0 = jnp.concatenate([s0_ref[k] for k in range(4)], axis=-1)
        S1 = jnp.concatenate([s1_ref[k] for k in range(4)], axis=-1)
    dg0 = d0_ref[:, 0:1]
    dg1 = d1_ref[:, 0:1]
    dg0 = jnp.where(dg0 == 0.0, 1.0, dg0)
    dg1 = jnp.where(dg1 == 0.0, 1.0, dg1)
    m0 = jnp.dot(S0, w0_ref[...], preferred_element_type=jnp.float32) / dg0
    m1 = jnp.dot(S1, w1_ref[...], preferred_element_type=jnp.float32) / dg1
    dn = (((1,), (1,)), ((), ()))
    sc0 = lax.dot_general(m0, a0_ref[...], dn,
                          preferred_element_type=jnp.float32)[:, :HEADS]
    sc1 = lax.dot_general(m1, a1_ref[...], dn,
                          preferred_element_type=jnp.float32)[:, :HEADS]
    sc0 = sc0 * _ATT_SCALE
    sc1 = sc1 * _ATT_SCALE
    mx = jnp.maximum(sc0, sc1)
    e0 = jnp.exp(sc0 - mx)
    e1 = jnp.exp(sc1 - mx)
    wt0 = jnp.mean(e0 / (e0 + e1), axis=1, keepdims=True)
    wt1 = 1.0 - wt0
    agg = wt0 * m0 + wt1 * m1
    hb = h_ref[...]
    h_up = jnp.dot(hb, ws_ref[...], preferred_element_type=jnp.float32) + agg + hb
    mu = jnp.mean(h_up, axis=1, keepdims=True)
    cen = h_up - mu
    var = jnp.mean(cen * cen, axis=1, keepdims=True)
    hn = cen / jnp.sqrt(var + 1e-5) * g_ref[...] + b_ref[...]
    o_ref[...] = jnp.where(hn >= 0.0, hn, 0.01 * hn)


def _update_call(partials, n, h, S0, D0, S1, D1, W0, W1, Ws, a0p, a1p, g2, b2):
    bn = 400
    if partials:
        s_spec = pl.BlockSpec((2, bn, D), lambda i: (0, i, 0))
    else:
        s_spec = pl.BlockSpec((4, bn, 32), lambda i: (0, i, 0))
    d_spec = pl.BlockSpec((bn, 16), lambda i: (i, 0))
    wspec = pl.BlockSpec((D, D), lambda i: (0, 0))
    aspec = pl.BlockSpec((8, D), lambda i: (0, 0))
    vspec = pl.BlockSpec((1, D), lambda i: (0, 0))
    return pl.pallas_call(
        functools.partial(_update_body, partials),
        grid=(n // bn,),
        in_specs=[pl.BlockSpec((bn, D), lambda i: (i, 0)),
                  s_spec, d_spec, s_spec, d_spec,
                  wspec, wspec, wspec, aspec, aspec, vspec, vspec],
        out_specs=pl.BlockSpec((bn, D), lambda i: (i, 0)),
        out_shape=jax.ShapeDtypeStruct((n, D), jnp.float32),
    )(h, S0, D0, S1, D1, W0, W1, Ws, a0p, a1p, g2, b2)


def _pad_edges(src, dst, n_dst, npad):
    pad_dst = n_dst + jnp.arange(EPAD - E, dtype=jnp.int32) % (npad - n_dst)
    src_p = jnp.concatenate(
        [src.astype(jnp.int32), jnp.zeros((EPAD - E,), jnp.int32)])
    dst_p = jnp.concatenate([dst.astype(jnp.int32), pad_dst])
    return src_p.reshape(EROWS, 1, BATCH), dst_p.reshape(EROWS, 1, BATCH)


def _pad_att(a):
    return jnp.pad(a, ((0, 8 - HEADS), (0, 0)))


def kernel(h_lncRNA, h_gene, h_disease, src_lg, dst_lg, src_dg, dst_dg,
           src_gd, dst_gd, src_ld, dst_ld, src_gl, dst_gl, src_dl, dst_dl,
           W_lg, W_dg, W_gd, W_ld, W_gl, W_dl, Ws_l, Ws_g, Ws_d,
           a_lg, a_dg, a_gd, a_ld, a_gl, a_dl,
           ln_g_l, ln_b_l, ln_g_g, ln_b_g, ln_g_d, ln_b_d):
    s_gd, d_gd = _pad_edges(src_gd, dst_gd, N_D, NPS)
    s_ld, d_ld = _pad_edges(src_ld, dst_ld, N_D, NPS)
    s_gl, d_gl = _pad_edges(src_gl, dst_gl, N_L, NPS)
    s_dl, d_dl = _pad_edges(src_dl, dst_dl, N_L, NPS)
    s_lg, d_lg = _pad_edges(src_lg, dst_lg, N_G, NPG)
    s_dg, d_dg = _pad_edges(src_dg, dst_dg, N_G, NPG)

    S_gd = _seg_full(h_gene, s_gd, d_gd)
    S_ld = _seg_full(h_lncRNA, s_ld, d_ld)
    S_gl = _seg_full(h_gene, s_gl, d_gl)
    S_dl = _seg_full(h_disease, s_dl, d_dl)
    S_lg = _seg_chunk(h_lncRNA.reshape(4 * N_L, 32), s_lg, d_lg)
    S_dg = _seg_chunk(h_disease.reshape(4 * N_D, 32), s_dg, d_dg)
    Dg_s, Dg_g = _deg_x6(jnp.stack([d_gd, d_ld, d_gl, d_dl]),
                         jnp.stack([d_lg, d_dg]))
    Dg_gd, Dg_ld, Dg_gl, Dg_dl = Dg_s[0], Dg_s[1], Dg_s[2], Dg_s[3]
    Dg_lg, Dg_dg = Dg_g[0], Dg_g[1]

    g_l, b_l = ln_g_l.reshape(1, D), ln_b_l.reshape(1, D)
    g_g, b_g = ln_g_g.reshape(1, D), ln_b_g.reshape(1, D)
    g_d, b_d = ln_g_d.reshape(1, D), ln_b_d.reshape(1, D)
    out_l = _update_call(True, N_L, h_lncRNA, S_gl, Dg_gl, S_dl, Dg_dl,
                         W_gl, W_dl, Ws_l, _pad_att(a_gl), _pad_att(a_dl),
                         g_l, b_l)
    out_g = _update_call(False, N_G, h_gene, S_lg, Dg_lg, S_dg, Dg_dg,
                         W_lg, W_dg, Ws_g, _pad_att(a_lg), _pad_att(a_dg),
                         g_g, b_g)
    out_d = _update_call(True, N_D, h_disease, S_gd, Dg_gd, S_ld, Dg_ld,
                         W_gd, W_ld, Ws_d, _pad_att(a_gd), _pad_att(a_ld),
                         g_d, b_d)
    return (out_l, out_g, out_d)


# speedup vs baseline: 1.1142x; 1.0201x over previous
import functools

import jax
import jax.numpy as jnp
from jax import lax
from jax.experimental import pallas as pl
from jax.experimental.pallas import tpu as pltpu
from jax.experimental.pallas import tpu_sc as plsc

D = 128
HEADS = 4
N_L = 10000
N_G = 50000
N_D = 10000
E = 100000

NSUB = 16
BATCH = 128
EPAD = 102400
EROWS = EPAD // BATCH
NPS = 10240
NPG = 50176


def _mesh():
    return plsc.VectorSubcoreMesh(core_axis_name="c", subcore_axis_name="s")


def _fill16(ref, n, val):
    @pl.loop(0, n)
    def _(i):
        ref[i, :] = val


def _seg_full(table, src3d, dst3d):
    rows_per = NPS // NSUB
    nbatch = EROWS // 32

    @functools.partial(
        pl.kernel,
        mesh=_mesh(),
        out_type=jax.ShapeDtypeStruct((2, NPS, D), jnp.float32),
        scratch_types=[
            pltpu.VMEM_SHARED((NPS, D), jnp.float32),
            pltpu.VMEM((nbatch, 1, BATCH), jnp.int32),
            pltpu.VMEM((nbatch, 1, BATCH), jnp.int32),
            pltpu.VMEM((BATCH, D), jnp.float32),
            pltpu.VMEM((64, D), jnp.float32),
            pltpu.SemaphoreType.DMA,
        ],
    )
    def k(tbl, src, dst, out_sum, acc, src_v, dst_v, rows_v, zb, sem):
        c = lax.axis_index("c")
        s = lax.axis_index("s")
        w = c * NSUB + s
        zero16 = jnp.zeros((16,), jnp.float32)

        @pl.loop(0, 64)
        def _(i):
            for j in range(D // 16):
                zb[i, pl.ds(j * 16, 16)] = zero16

        r0 = s * rows_per

        @pl.loop(0, rows_per // 64)
        def _(t):
            pltpu.sync_copy(zb, acc.at[pl.ds(r0 + t * 64, 64)])

        pltpu.sync_copy(src.at[pl.ds(w * nbatch, nbatch)], src_v)
        pltpu.sync_copy(dst.at[pl.ds(w * nbatch, nbatch)], dst_v)
        plsc.subcore_barrier()

        @pl.loop(0, nbatch)
        def _(j):
            pltpu.async_copy(tbl.at[src_v.at[j, 0]], rows_v, sem).wait()
            pltpu.sync_copy(rows_v, acc.at[dst_v.at[j, 0]], add=True)

        plsc.subcore_barrier()
        pltpu.sync_copy(acc.at[pl.ds(r0, rows_per)],
                        out_sum.at[c, pl.ds(r0, rows_per)])

    return k(table, src3d, dst3d)


def _seg_chunk(table4, src3d, dst3d):
    rows_per = NPG // NSUB
    nbatch = EROWS // NSUB

    @functools.partial(
        pl.kernel,
        mesh=_mesh(),
        compiler_params=pltpu.CompilerParams(use_tc_tiling_on_sc=False),
        out_type=jax.ShapeDtypeStruct((4, NPG, 32), jnp.float32),
        scratch_types=[
            pltpu.VMEM_SHARED((NPG, 32), jnp.float32),
            pltpu.VMEM((nbatch, 1, BATCH), jnp.int32),
            pltpu.VMEM((nbatch, 1, BATCH), jnp.int32),
            pltpu.VMEM((nbatch, 1, BATCH), jnp.int32),
            pltpu.VMEM((BATCH, 32), jnp.float32),
            pltpu.VMEM((64, 32), jnp.float32),
            pltpu.SemaphoreType.DMA,
        ],
    )
    def k(tbl, src, dst, out_s, acc, src_v, sidx_v, dst_v, rows_v, zb32, sem):
        c = lax.axis_index("c")
        s = lax.axis_index("s")
        zero16 = jnp.zeros((16,), jnp.float32)

        @pl.loop(0, 64)
        def _(i):
            zb32[i, pl.ds(0, 16)] = zero16
            zb32[i, pl.ds(16, 16)] = zero16

        r0 = s * rows_per
        pltpu.sync_copy(src.at[pl.ds(s * nbatch, nbatch)], src_v)
        pltpu.sync_copy(dst.at[pl.ds(s * nbatch, nbatch)], dst_v)

        for k2 in range(2):
            chunk = c * 2 + k2

            @pl.loop(0, rows_per // 64)
            def _(t):
                pltpu.sync_copy(zb32, acc.at[pl.ds(r0 + t * 64, 64)])

            @pl.loop(0, nbatch)
            def _(i):
                for j in range(BATCH // 16):
                    sidx_v[i, 0, pl.ds(j * 16, 16)] = (
                        src_v[i, 0, pl.ds(j * 16, 16)] * 4 + chunk)

            plsc.subcore_barrier()

            @pl.loop(0, nbatch)
            def _(j):
                pltpu.async_copy(tbl.at[sidx_v.at[j, 0]], rows_v, sem).wait()
                pltpu.sync_copy(rows_v, acc.at[dst_v.at[j, 0]], add=True)

            plsc.subcore_barrier()
            pltpu.sync_copy(acc.at[pl.ds(r0, rows_per)],
                            out_s.at[chunk, pl.ds(r0, rows_per)])

    return k(table4, src3d, dst3d)


def _deg_x6(dsts_small, dsts_gene):
    rps = NPS // NSUB
    rpg = NPG // NSUB
    nbatch = EROWS // NSUB

    @functools.partial(
        pl.kernel,
        mesh=_mesh(),
        compiler_params=pltpu.CompilerParams(use_tc_tiling_on_sc=False),
        out_type=(jax.ShapeDtypeStruct((4, NPS, 16), jnp.float32),
                  jax.ShapeDtypeStruct((2, NPG, 16), jnp.float32)),
        scratch_types=[
            pltpu.VMEM_SHARED((NPG, 16), jnp.float32),
            pltpu.VMEM((nbatch, 1, BATCH), jnp.int32),
            pltpu.VMEM((BATCH, 16), jnp.float32),
            pltpu.VMEM((16, 16), jnp.float32),
        ],
    )
    def k(dsml, dgen, out_s, out_g, dacc, dst_v, ones_v, zb16):
        c = lax.axis_index("c")
        s = lax.axis_index("s")
        _fill16(zb16, 16, jnp.zeros((16,), jnp.float32))
        _fill16(ones_v, BATCH, jnp.ones((16,), jnp.float32))

        for (inref, outref, nrel_pc, rows_per) in (
                (dsml, out_s, 2, rps), (dgen, out_g, 1, rpg)):
            r0 = s * rows_per
            for rr in range(nrel_pc):
                rel = c * nrel_pc + rr

                @pl.loop(0, rows_per // 16)
                def _(t):
                    pltpu.sync_copy(zb16, dacc.at[pl.ds(r0 + t * 16, 16)])

                pltpu.sync_copy(inref.at[rel, pl.ds(s * nbatch, nbatch)],
                                dst_v)
                plsc.subcore_barrier()

                @pl.loop(0, nbatch)
                def _(j):
                    pltpu.sync_copy(ones_v, dacc.at[dst_v.at[j, 0]],
                                    add=True)

                plsc.subcore_barrier()
                pltpu.sync_copy(dacc.at[pl.ds(r0, rows_per)],
                                outref.at[rel, pl.ds(r0, rows_per)])

    return k(dsts_small, dsts_gene)


_ATT_SCALE = 1.0 / (float(D) ** 0.5 * 0.5)


def _update_body(partials, h_ref, s0_ref, d0_ref, s1_ref, d1_ref,
                 w0_ref, w1_ref, ws_ref, a0_ref, a1_ref, g_ref, b_ref, o_ref):
    if partials:
        S0 = s0_ref[0] + s0_ref[1]
        S1 = s1_ref[0] + s1_ref[1]
    else:
        S0 = jnp.concatenate([s0_ref[k] for k in range(4)], axis=-1)
        S1 = jnp.concatenate([s1_ref[k] for k in range(4)], axis=-1)
    dg0 = d0_ref[:, 0:1]
    dg1 = d1_ref[:, 0:1]
    dg0 = jnp.where(dg0 == 0.0, 1.0, dg0)
    dg1 = jnp.where(dg1 == 0.0, 1.0, dg1)
    m0 = jnp.dot(S0, w0_ref[...], preferred_element_type=jnp.float32) / dg0
    m1 = jnp.dot(S1, w1_ref[...], preferred_element_type=jnp.float32) / dg1
    dn = (((1,), (1,)), ((), ()))
    sc0 = lax.dot_general(m0, a0_ref[...], dn,
                          preferred_element_type=jnp.float32)[:, :HEADS]
    sc1 = lax.dot_general(m1, a1_ref[...], dn,
                          preferred_element_type=jnp.float32)[:, :HEADS]
    sc0 = sc0 * _ATT_SCALE
    sc1 = sc1 * _ATT_SCALE
    mx = jnp.maximum(sc0, sc1)
    e0 = jnp.exp(sc0 - mx)
    e1 = jnp.exp(sc1 - mx)
    wt0 = jnp.mean(e0 / (e0 + e1), axis=1, keepdims=True)
    wt1 = 1.0 - wt0
    agg = wt0 * m0 + wt1 * m1
    hb = h_ref[...]
    h_up = jnp.dot(hb, ws_ref[...], preferred_element_type=jnp.float32) + agg + hb
    mu = jnp.mean(h_up, axis=1, keepdims=True)
    cen = h_up - mu
    var = jnp.mean(cen * cen, axis=1, keepdims=True)
    hn = cen / jnp.sqrt(var + 1e-5) * g_ref[...] + b_ref[...]
    o_ref[...] = jnp.where(hn >= 0.0, hn, 0.01 * hn)


def _update_call(partials, n, h, S0, D0, S1, D1, W0, W1, Ws, a0p, a1p, g2, b2):
    bn = 400
    if partials:
        s_spec = pl.BlockSpec((2, bn, D), lambda i: (0, i, 0))
    else:
        s_spec = pl.BlockSpec((4, bn, 32), lambda i: (0, i, 0))
    d_spec = pl.BlockSpec((bn, 16), lambda i: (i, 0))
    wspec = pl.BlockSpec((D, D), lambda i: (0, 0))
    aspec = pl.BlockSpec((8, D), lambda i: (0, 0))
    vspec = pl.BlockSpec((1, D), lambda i: (0, 0))
    return pl.pallas_call(
        functools.partial(_update_body, partials),
        grid=(n // bn,),
        in_specs=[pl.BlockSpec((bn, D), lambda i: (i, 0)),
                  s_spec, d_spec, s_spec, d_spec,
                  wspec, wspec, wspec, aspec, aspec, vspec, vspec],
        out_specs=pl.BlockSpec((bn, D), lambda i: (i, 0)),
        out_shape=jax.ShapeDtypeStruct((n, D), jnp.float32),
    )(h, S0, D0, S1, D1, W0, W1, Ws, a0p, a1p, g2, b2)


def _pad_edges(src, dst, n_dst, npad):
    pad_dst = n_dst + jnp.arange(EPAD - E, dtype=jnp.int32) % (npad - n_dst)
    src_p = jnp.concatenate(
        [src.astype(jnp.int32), jnp.zeros((EPAD - E,), jnp.int32)])
    dst_p = jnp.concatenate([dst.astype(jnp.int32), pad_dst])
    return src_p.reshape(EROWS, 1, BATCH), dst_p.reshape(EROWS, 1, BATCH)


def _pad_att(a):
    return jnp.pad(a, ((0, 8 - HEADS), (0, 0)))


def kernel(h_lncRNA, h_gene, h_disease, src_lg, dst_lg, src_dg, dst_dg,
           src_gd, dst_gd, src_ld, dst_ld, src_gl, dst_gl, src_dl, dst_dl,
           W_lg, W_dg, W_gd, W_ld, W_gl, W_dl, Ws_l, Ws_g, Ws_d,
           a_lg, a_dg, a_gd, a_ld, a_gl, a_dl,
           ln_g_l, ln_b_l, ln_g_g, ln_b_g, ln_g_d, ln_b_d):
    s_gd, d_gd = _pad_edges(src_gd, dst_gd, N_D, NPS)
    s_ld, d_ld = _pad_edges(src_ld, dst_ld, N_D, NPS)
    s_gl, d_gl = _pad_edges(src_gl, dst_gl, N_L, NPS)
    s_dl, d_dl = _pad_edges(src_dl, dst_dl, N_L, NPS)
    s_lg, d_lg = _pad_edges(src_lg, dst_lg, N_G, NPG)
    s_dg, d_dg = _pad_edges(src_dg, dst_dg, N_G, NPG)

    S_gd = _seg_full(h_gene, s_gd, d_gd)
    S_ld = _seg_full(h_lncRNA, s_ld, d_ld)
    S_gl = _seg_full(h_gene, s_gl, d_gl)
    S_dl = _seg_full(h_disease, s_dl, d_dl)
    S_lg = _seg_chunk(h_lncRNA.reshape(4 * N_L, 32), s_lg, d_lg)
    S_dg = _seg_chunk(h_disease.reshape(4 * N_D, 32), s_dg, d_dg)
    Dg_s, Dg_g = _deg_x6(jnp.stack([d_gd, d_ld, d_gl, d_dl]),
                         jnp.stack([d_lg, d_dg]))
    Dg_gd, Dg_ld, Dg_gl, Dg_dl = Dg_s[0], Dg_s[1], Dg_s[2], Dg_s[3]
    Dg_lg, Dg_dg = Dg_g[0], Dg_g[1]

    g_l, b_l = ln_g_l.reshape(1, D), ln_b_l.reshape(1, D)
    g_g, b_g = ln_g_g.reshape(1, D), ln_b_g.reshape(1, D)
    g_d, b_d = ln_g_d.reshape(1, D), ln_b_d.reshape(1, D)
    out_l = _update_call(True, N_L, h_lncRNA, S_gl, Dg_gl, S_dl, Dg_dl,
                         W_gl, W_dl, Ws_l, _pad_att(a_gl), _pad_att(a_dl),
                         g_l, b_l)
    out_g = _update_call(False, N_G, h_gene, S_lg, Dg_lg, S_dg, Dg_dg,
                         W_lg, W_dg, Ws_g, _pad_att(a_lg), _pad_att(a_dg),
                         g_g, b_g)
    out_d = _update_call(True, N_D, h_disease, S_gd, Dg_gd, S_ld, Dg_ld,
                         W_gd, W_ld, Ws_d, _pad_att(a_gd), _pad_att(a_ld),
                         g_d, b_d)
    return (out_l, out_g, out_d)
